# baseline (speedup 1.0000x reference)
import functools

import jax
import jax.numpy as jnp
from jax import lax
from jax.experimental import pallas as pl
from jax.experimental.pallas import tpu as pltpu
from jax.experimental.pallas import tpu_sc as plsc

_N = 16384
_NSUB = 16
_CHUNK = _N // _NSUB
_VECS = _CHUNK // 16

_mesh = plsc.VectorSubcoreMesh(
    core_axis_name="c", subcore_axis_name="s", num_cores=1
)


@functools.partial(
    pl.kernel,
    out_type=jax.ShapeDtypeStruct((16,), jnp.float32),
    mesh=_mesh,
    scratch_types=[
        pltpu.VMEM((2 * _CHUNK,), jnp.float32),
        pltpu.VMEM((_CHUNK,), jnp.int32),
        pltpu.VMEM((_CHUNK,), jnp.int32),
        pltpu.VMEM((128,), jnp.float32),
        pltpu.VMEM((_NSUB, 128), jnp.float32),
        pltpu.VMEM_SHARED((_NSUB, 128), jnp.float32),
    ],
)
def _diff_eopp_sc(yp_hbm, s_hbm, yg_hbm, out_hbm,
                  yp_v, s_v, y_v, part_v, all_v, shared):
    sid = lax.axis_index("s")
    base = sid * _CHUNK
    pltpu.sync_copy(yp_hbm.at[pl.ds(base * 2, 2 * _CHUNK)], yp_v)
    pltpu.sync_copy(s_hbm.at[pl.ds(base, _CHUNK)], s_v)
    pltpu.sync_copy(yg_hbm.at[pl.ds(base, _CHUNK)], y_v)

    zero = jnp.zeros((16,), jnp.float32)
    lane = lax.iota(jnp.int32, 16)
    deint_idx = (2 * lane + 1) % 16
    low_half = lane < 8

    def body(i, carry):
        s0, c0, s1, c1 = carry
        off = i * 16
        sv = s_v[pl.ds(off, 16)]
        yv = y_v[pl.ds(off, 16)]
        a = yp_v[pl.ds(2 * off, 16)]
        b = yp_v[pl.ds(2 * off + 16, 16)]
        ga = a.at[deint_idx].get(mode="promise_in_bounds")
        gb = b.at[deint_idx].get(mode="promise_in_bounds")
        yp = jnp.where(low_half, ga, gb)
        m = yv == 1
        m0 = jnp.where(jnp.logical_and(m, sv == 0), 1.0, 0.0)
        m1 = jnp.where(jnp.logical_and(m, sv == 1), 1.0, 0.0)
        return (s0 + yp * m0, c0 + m0, s1 + yp * m1, c1 + m1)

    s0, c0, s1, c1 = lax.fori_loop(0, _VECS, body, (zero, zero, zero, zero))

    part_v[pl.ds(0, 16)] = s0
    part_v[pl.ds(16, 16)] = c0
    part_v[pl.ds(32, 16)] = s1
    part_v[pl.ds(48, 16)] = c1
    part_v[pl.ds(64, 16)] = zero
    pltpu.sync_copy(part_v, shared.at[sid])
    plsc.subcore_barrier()

    @pl.when(sid == 0)
    def _():
        pltpu.sync_copy(shared, all_v)
        acc = [zero, zero, zero, zero]
        for j in range(_NSUB):
            for q in range(4):
                acc[q] = acc[q] + all_v[j, pl.ds(16 * q, 16)]

        def lanesum(v):
            for d in (8, 4, 2, 1):
                v = v + v.at[lane ^ d].get(mode="promise_in_bounds")
            return v

        mean0 = lanesum(acc[0]) / lanesum(acc[1])
        mean1 = lanesum(acc[2]) / lanesum(acc[3])
        res = jnp.abs(mean0 - mean1)
        part_v[pl.ds(0, 16)] = res
        pltpu.sync_copy(part_v.at[pl.ds(0, 16)], out_hbm)


def kernel(y_pred, s, y_gt):
    out = _diff_eopp_sc(y_pred.reshape(-1), s, y_gt)
    return out[0]


# speedup vs baseline: 1.0404x; 1.0404x over previous
import functools

import jax
import jax.numpy as jnp
from jax import lax
from jax.experimental import pallas as pl
from jax.experimental.pallas import tpu as pltpu
from jax.experimental.pallas import tpu_sc as plsc

_N = 16384
_NSUB = 16
_CHUNK = _N // _NSUB
_VECS = _CHUNK // 16

_mesh = plsc.VectorSubcoreMesh(
    core_axis_name="c", subcore_axis_name="s", num_cores=1
)


@functools.partial(
    pl.kernel,
    out_type=jax.ShapeDtypeStruct((16,), jnp.float32),
    mesh=_mesh,
    scratch_types=[
        pltpu.VMEM((2 * _CHUNK,), jnp.float32),
        pltpu.VMEM((_CHUNK,), jnp.int32),
        pltpu.VMEM((_CHUNK,), jnp.int32),
        pltpu.VMEM((128,), jnp.float32),
        pltpu.VMEM((_NSUB, 128), jnp.float32),
        pltpu.VMEM_SHARED((_NSUB, 128), jnp.float32),
        pltpu.SemaphoreType.DMA,
        pltpu.SemaphoreType.DMA,
        pltpu.SemaphoreType.DMA,
    ],
)
def _diff_eopp_sc(yp_hbm, s_hbm, yg_hbm, out_hbm,
                  yp_v, s_v, y_v, part_v, all_v, shared,
                  sem0, sem1, sem2):
    sid = lax.axis_index("s")
    base = sid * _CHUNK
    cp0 = pltpu.async_copy(yp_hbm.at[pl.ds(base * 2, 2 * _CHUNK)], yp_v, sem0)
    cp1 = pltpu.async_copy(s_hbm.at[pl.ds(base, _CHUNK)], s_v, sem1)
    cp2 = pltpu.async_copy(yg_hbm.at[pl.ds(base, _CHUNK)], y_v, sem2)
    cp0.wait()
    cp1.wait()
    cp2.wait()

    zero = jnp.zeros((16,), jnp.float32)
    lane = lax.iota(jnp.int32, 16)
    deint_idx = (2 * lane + 1) % 16
    low_half = lane < 8

    def body(i, carry):
        s0, c0, s1, c1 = carry
        for k in range(2):
            off = i * 32 + k * 16
            sv = s_v[pl.ds(off, 16)]
            yv = y_v[pl.ds(off, 16)]
            a = yp_v[pl.ds(2 * off, 16)]
            b = yp_v[pl.ds(2 * off + 16, 16)]
            ga = a.at[deint_idx].get(mode="promise_in_bounds")
            gb = b.at[deint_idx].get(mode="promise_in_bounds")
            yp = jnp.where(low_half, ga, gb)
            m = yv == 1
            m0 = jnp.where(jnp.logical_and(m, sv == 0), 1.0, 0.0)
            m1 = jnp.where(jnp.logical_and(m, sv == 1), 1.0, 0.0)
            s0, c0, s1, c1 = s0 + yp * m0, c0 + m0, s1 + yp * m1, c1 + m1
        return (s0, c0, s1, c1)

    s0, c0, s1, c1 = plsc.parallel_loop(
        0, _VECS // 2, unroll=2, carry=(zero, zero, zero, zero)
    )(body)

    part_v[pl.ds(0, 16)] = s0
    part_v[pl.ds(16, 16)] = c0
    part_v[pl.ds(32, 16)] = s1
    part_v[pl.ds(48, 16)] = c1
    pltpu.sync_copy(part_v, shared.at[sid])
    plsc.subcore_barrier()

    @pl.when(sid == 0)
    def _():
        pltpu.sync_copy(shared, all_v)
        acc = [zero, zero, zero, zero]
        for j in range(_NSUB):
            for q in range(4):
                acc[q] = acc[q] + all_v[j, pl.ds(16 * q, 16)]

        def lanesum(v):
            for d in (8, 4, 2, 1):
                v = v + v.at[lane ^ d].get(mode="promise_in_bounds")
            return v

        mean0 = lanesum(acc[0]) / lanesum(acc[1])
        mean1 = lanesum(acc[2]) / lanesum(acc[3])
        res = jnp.abs(mean0 - mean1)
        part_v[pl.ds(0, 16)] = res
        pltpu.sync_copy(part_v.at[pl.ds(0, 16)], out_hbm)


def kernel(y_pred, s, y_gt):
    out = _diff_eopp_sc(y_pred.reshape(-1), s, y_gt)
    return out[0]
